# Initial kernel scaffold; baseline (speedup 1.0000x reference)
import jax
import jax.numpy as jnp
from jax.experimental import pallas as pl


def kernel(x, edge_index, time_step, Wt1, bt1, Wt2, bt2, gamma, beta, Win, msgW1, msgb1, msgW2, msgb2, featW1, featb1, featW2, featb2, Wout):
    raise NotImplementedError("write your pallas kernel here")


# speedup vs baseline: 2.6630x; 2.6630x over previous
import functools

import jax
import jax.numpy as jnp
from jax import lax
from jax.experimental import pallas as pl
from jax.experimental.pallas import tpu as pltpu
from jax.experimental.pallas import tpu_sc as plsc

N = 10000
E = 320000
D = 128
MD = 32
L = 4
DOUT = 128

NC = 2
NS = 16
NW = NC * NS
EPW = E // NW
K = 80
NCH = EPW // K
NPS = N // NS

BN = 1000
BE = 2000


def _softsign(v):
    return v / (1.0 + jnp.abs(v))


@functools.lru_cache(maxsize=None)
def _make_sc_gather():
    mesh = plsc.VectorSubcoreMesh(core_axis_name="c", subcore_axis_name="s")

    @functools.partial(
        pl.kernel,
        mesh=mesh,
        compiler_params=pltpu.CompilerParams(use_tc_tiling_on_sc=False),
        out_type=jax.ShapeDtypeStruct((E, MD), jnp.float32),
        scratch_types=[
            pltpu.VMEM((K,), jnp.int32),
            pltpu.VMEM((K,), jnp.int32),
            pltpu.VMEM((K, MD), jnp.float32),
            pltpu.VMEM((K, MD), jnp.float32),
            pltpu.VMEM((K, MD), jnp.float32),
            pltpu.SemaphoreType.DMA,
            pltpu.SemaphoreType.DMA,
        ],
    )
    def sc_gather(a_hbm, b_hbm, row_hbm, col_hbm, s_hbm,
                  rid, cid, abuf, bbuf, sbuf, sema, semb):
        wid = lax.axis_index("s") * NC + lax.axis_index("c")
        base = wid * EPW

        def chunk_body(j, carry):
            off = base + j * K
            pltpu.sync_copy(row_hbm.at[pl.ds(off, K)], rid)
            pltpu.sync_copy(col_hbm.at[pl.ds(off, K)], cid)
            ca = pltpu.async_copy(a_hbm.at[rid], abuf, sema)
            cb = pltpu.async_copy(b_hbm.at[cid], bbuf, semb)
            ca.wait()
            cb.wait()

            def add_row(i, c2):
                for h in range(MD // 16):
                    sl = pl.ds(h * 16, 16)
                    sbuf[i, sl] = abuf[i, sl] + bbuf[i, sl]
                return c2

            lax.fori_loop(0, K, add_row, 0)
            pltpu.sync_copy(sbuf, s_hbm.at[pl.ds(off, K)])
            return carry

        lax.fori_loop(0, NCH, chunk_body, 0)

    return sc_gather


@functools.lru_cache(maxsize=None)
def _make_sc_scatter():
    mesh = plsc.VectorSubcoreMesh(core_axis_name="c", subcore_axis_name="s")

    @functools.partial(
        pl.kernel,
        mesh=mesh,
        compiler_params=pltpu.CompilerParams(use_tc_tiling_on_sc=False),
        out_type=jax.ShapeDtypeStruct((NC, N, MD), jnp.float32),
        scratch_types=[
            pltpu.VMEM((K,), jnp.int32),
            pltpu.VMEM((K, MD), jnp.float32),
            pltpu.VMEM((NPS, MD), jnp.float32),
            pltpu.VMEM_SHARED((N, MD), jnp.float32),
        ],
    )
    def sc_scatter(msg_hbm, row_hbm, aggp_hbm, rid, mbuf, zbuf, agg_sh):
        cidx = lax.axis_index("c")
        sidx = lax.axis_index("s")
        wid = sidx * NC + cidx

        zero = jnp.zeros((16,), jnp.float32)

        def zrow(i, c2):
            for h in range(MD // 16):
                zbuf[i, pl.ds(h * 16, 16)] = zero
            return c2

        lax.fori_loop(0, NPS, zrow, 0)
        pltpu.sync_copy(zbuf, agg_sh.at[pl.ds(sidx * NPS, NPS)])
        plsc.subcore_barrier()

        base = wid * EPW

        def chunk_body(j, carry):
            off = base + j * K
            pltpu.sync_copy(row_hbm.at[pl.ds(off, K)], rid)
            pltpu.sync_copy(msg_hbm.at[pl.ds(off, K)], mbuf)
            pltpu.sync_copy(mbuf, agg_sh.at[rid], add=True)
            return carry

        lax.fori_loop(0, NCH, chunk_body, 0)
        plsc.subcore_barrier()
        pltpu.sync_copy(agg_sh.at[pl.ds(sidx * NPS, NPS)],
                        aggp_hbm.at[cidx, pl.ds(sidx * NPS, NPS)])

    return sc_scatter


def _temb_in_kernel(ts, wt1, bt1, wt2, bt2):
    h = ts * wt1 + bt1
    h = h * jax.nn.sigmoid(h)
    return jnp.dot(h, wt2, preferred_element_type=jnp.float32) + bt2


def _tc_pre_kernel(x_ref, ts_ref, wt1_ref, bt1_ref, wt2_ref, bt2_ref,
                   gx_ref, gt_ref, bx_ref, btc_ref, winx_ref, wint_ref,
                   w1a_ref, w1b_ref, b1_ref,
                   feats_ref, a_ref, b_ref):
    temb = _temb_in_kernel(ts_ref[...], wt1_ref[...], bt1_ref[...],
                           wt2_ref[...], bt2_ref[...])
    xb = x_ref[...]
    sx = jnp.sum(xb, axis=1, keepdims=True)
    st = jnp.sum(temb, axis=1, keepdims=True)
    mu = (sx + st) / float(D + MD)
    xc = xb - mu
    tc = temb - mu
    var = (jnp.sum(xc * xc, axis=1, keepdims=True)
           + jnp.sum(tc * tc, axis=1, keepdims=True)) / float(D + MD)
    rstd = lax.rsqrt(var + 1e-3)
    fx = xc * rstd * gx_ref[...] + bx_ref[...]
    ft = tc * rstd * gt_ref[...] + btc_ref[...]
    feats = (jnp.dot(fx, winx_ref[...], preferred_element_type=jnp.float32)
             + jnp.dot(ft, wint_ref[...], preferred_element_type=jnp.float32))
    feats_ref[...] = feats
    a_ref[...] = jnp.dot(feats, w1a_ref[...],
                         preferred_element_type=jnp.float32) + b1_ref[...]
    b_ref[...] = jnp.dot(feats, w1b_ref[...],
                         preferred_element_type=jnp.float32)


def _tc_pre(x, ts32, wt1, bt1, wt2, bt2, gx, gt, bx, btc, winx, wint,
            w1a, w1b, b1):
    full = lambda shp: pl.BlockSpec(shp, lambda i: (0,) * len(shp))
    return pl.pallas_call(
        _tc_pre_kernel,
        grid=(N // BN,),
        in_specs=[
            pl.BlockSpec((BN, D), lambda i: (i, 0)),
            full((1, MD)), full((1, MD)), full((1, MD)),
            full((MD, MD)), full((1, MD)),
            full((1, D)), full((1, MD)), full((1, D)), full((1, MD)),
            full((D, MD)), full((MD, MD)),
            full((MD, MD)), full((MD, MD)), full((1, MD)),
        ],
        out_specs=[
            pl.BlockSpec((BN, MD), lambda i: (i, 0)),
            pl.BlockSpec((BN, MD), lambda i: (i, 0)),
            pl.BlockSpec((BN, MD), lambda i: (i, 0)),
        ],
        out_shape=[
            jax.ShapeDtypeStruct((N, MD), jnp.float32),
            jax.ShapeDtypeStruct((N, MD), jnp.float32),
            jax.ShapeDtypeStruct((N, MD), jnp.float32),
        ],
    )(x, ts32, wt1, bt1, wt2, bt2, gx, gt, bx, btc, winx, wint, w1a, w1b, b1)


def _tc_msg_kernel(s_ref, w2_ref, b2_ref, msg_ref):
    m1 = jax.nn.sigmoid(s_ref[...])
    z = jnp.dot(m1, w2_ref[...], preferred_element_type=jnp.float32) + b2_ref[...]
    msg_ref[...] = _softsign(z)


def _tc_msg(s, w2, b2):
    full = lambda shp: pl.BlockSpec(shp, lambda i: (0,) * len(shp))
    return pl.pallas_call(
        _tc_msg_kernel,
        grid=(E // BE,),
        in_specs=[
            pl.BlockSpec((BE, MD), lambda i: (i, 0)),
            full((MD, MD)), full((1, MD)),
        ],
        out_specs=pl.BlockSpec((BE, MD), lambda i: (i, 0)),
        out_shape=jax.ShapeDtypeStruct((E, MD), jnp.float32),
    )(s, w2, b2)


def _tc_update_kernel(last, f_ref, aggp_ref, ts_ref, wt1_ref, bt1_ref,
                      wt2_ref, bt2_ref, fa_ref, fb_ref, fc_ref, fb1_ref,
                      fw2_ref, fb2_ref, wx_ref, wy_ref, by_ref, *out_refs):
    temb = _temb_in_kernel(ts_ref[...], wt1_ref[...], bt1_ref[...],
                           wt2_ref[...], bt2_ref[...])
    agg = aggp_ref[0] + aggp_ref[1]
    gs = jax.nn.sigmoid(f_ref[...])
    ga = jax.nn.sigmoid(agg)
    gt = jax.nn.sigmoid(temb)
    g1 = jax.nn.sigmoid(
        jnp.dot(gs, fa_ref[...], preferred_element_type=jnp.float32)
        + jnp.dot(ga, fb_ref[...], preferred_element_type=jnp.float32)
        + jnp.dot(gt, fc_ref[...], preferred_element_type=jnp.float32)
        + fb1_ref[...])
    f2 = _softsign(jnp.dot(g1, fw2_ref[...],
                           preferred_element_type=jnp.float32) + fb2_ref[...])
    if last:
        out_refs[0][...] = jnp.dot(f2, wx_ref[...],
                                   preferred_element_type=jnp.float32)
    else:
        out_refs[0][...] = f2
        out_refs[1][...] = jnp.dot(f2, wx_ref[...],
                                   preferred_element_type=jnp.float32) + by_ref[...]
        out_refs[2][...] = jnp.dot(f2, wy_ref[...],
                                   preferred_element_type=jnp.float32)


def _tc_update(last, feats, aggp, ts32, wt1, bt1, wt2, bt2,
               fa, fb, fc, fb1, fw2, fb2, wx, wy, by):
    full = lambda shp: pl.BlockSpec(shp, lambda i: (0,) * len(shp))
    if last:
        out_specs = [pl.BlockSpec((BN, DOUT), lambda i: (i, 0))]
        out_shape = [jax.ShapeDtypeStruct((N, DOUT), jnp.float32)]
        wx_spec = full((MD, DOUT))
    else:
        out_specs = [pl.BlockSpec((BN, MD), lambda i: (i, 0))] * 3
        out_shape = [jax.ShapeDtypeStruct((N, MD), jnp.float32)] * 3
        wx_spec = full((MD, MD))
    res = pl.pallas_call(
        functools.partial(_tc_update_kernel, last),
        grid=(N // BN,),
        in_specs=[
            pl.BlockSpec((BN, MD), lambda i: (i, 0)),
            pl.BlockSpec((NC, BN, MD), lambda i: (0, i, 0)),
            full((1, MD)), full((1, MD)), full((1, MD)),
            full((MD, MD)), full((1, MD)),
            full((MD, MD)), full((MD, MD)), full((MD, MD)), full((1, MD)),
            full((MD, MD)), full((1, MD)),
            wx_spec, full((MD, MD)), full((1, MD)),
        ],
        out_specs=out_specs,
        out_shape=out_shape,
    )(feats, aggp, ts32, wt1, bt1, wt2, bt2,
      fa, fb, fc, fb1, fw2, fb2, wx, wy, by)
    return res


def kernel(x, edge_index, time_step, Wt1, bt1, Wt2, bt2, gamma, beta, Win,
           msgW1, msgb1, msgW2, msgb2, featW1, featb1, featW2, featb2, Wout):
    ei = edge_index.astype(jnp.int32)
    row = ei[0]
    col = ei[1]
    ts32 = jnp.full((1, MD), time_step, jnp.float32)

    wt1 = Wt1.reshape(1, MD)
    bt1r = bt1.reshape(1, MD)
    bt2r = bt2.reshape(1, MD)
    gx = gamma[:D].reshape(1, D)
    gt = gamma[D:].reshape(1, MD)
    bx = beta[:D].reshape(1, D)
    btc = beta[D:].reshape(1, MD)
    winx = Win[:D]
    wint = Win[D:]

    w1a = [msgW1[i, :MD] for i in range(L)]
    w1b = [msgW1[i, MD:] for i in range(L)]
    b1 = [msgb1[i].reshape(1, MD) for i in range(L)]
    w2 = [msgW2[i] for i in range(L)]
    b2 = [msgb2[i].reshape(1, MD) for i in range(L)]
    fa = [featW1[i, :MD] for i in range(L)]
    fb = [featW1[i, MD:2 * MD] for i in range(L)]
    fc = [featW1[i, 2 * MD:] for i in range(L)]
    fb1 = [featb1[i].reshape(1, MD) for i in range(L)]
    fw2 = [featW2[i] for i in range(L)]
    fb2 = [featb2[i].reshape(1, MD) for i in range(L)]

    feats, A, B = _tc_pre(x, ts32, wt1, bt1r, Wt2, bt2r, gx, gt, bx, btc,
                          winx, wint, w1a[0], w1b[0], b1[0])

    sc_gather = _make_sc_gather()
    sc_scatter = _make_sc_scatter()
    out = None
    for i in range(L):
        s = sc_gather(A, B, row, col)
        msg = _tc_msg(s, w2[i], b2[i])
        aggp = sc_scatter(msg, row)
        last = i == L - 1
        if last:
            (out,) = _tc_update(True, feats, aggp, ts32, wt1, bt1r, Wt2,
                                bt2r, fa[i], fb[i], fc[i], fb1[i], fw2[i],
                                fb2[i], Wout, fw2[i], fb2[i])
        else:
            feats, A, B = _tc_update(False, feats, aggp, ts32, wt1, bt1r,
                                     Wt2, bt2r, fa[i], fb[i], fc[i], fb1[i],
                                     fw2[i], fb2[i], w1a[i + 1], w1b[i + 1],
                                     b1[i + 1])
    return out


# speedup vs baseline: 4.4323x; 1.6644x over previous
import functools

import jax
import jax.numpy as jnp
from jax import lax
from jax.experimental import pallas as pl
from jax.experimental.pallas import tpu as pltpu
from jax.experimental.pallas import tpu_sc as plsc

N = 10000
E = 320000
D = 128
MD = 32
L = 4
DOUT = 128

NC = 2
NS = 16
NW = NC * NS
EPW = E // NW
K = 80
MEGA = 400
SUB = MEGA // K
NMEGA = EPW // MEGA
NPS = N // NS

BN = 1000
BE = 2000


def _softsign(v):
    return v / (1.0 + jnp.abs(v))


@functools.lru_cache(maxsize=None)
def _make_sc_gather():
    mesh = plsc.VectorSubcoreMesh(core_axis_name="c", subcore_axis_name="s")

    @functools.partial(
        pl.kernel,
        mesh=mesh,
        compiler_params=pltpu.CompilerParams(use_tc_tiling_on_sc=False),
        out_type=jax.ShapeDtypeStruct((E, MD), jnp.float32),
        scratch_types=[
            pltpu.VMEM((EPW,), jnp.int32),
            pltpu.VMEM((EPW,), jnp.int32),
            pltpu.VMEM((MEGA, MD), jnp.float32),
            pltpu.VMEM((MEGA, MD), jnp.float32),
            pltpu.VMEM((MEGA, MD), jnp.float32),
            pltpu.VMEM((MEGA, MD), jnp.float32),
            pltpu.VMEM((MEGA, MD), jnp.float32),
            pltpu.VMEM((MEGA, MD), jnp.float32),
            pltpu.SemaphoreType.DMA,
            pltpu.SemaphoreType.DMA,
            pltpu.SemaphoreType.DMA,
            pltpu.SemaphoreType.DMA,
            pltpu.SemaphoreType.DMA,
            pltpu.SemaphoreType.DMA,
        ],
    )
    def sc_gather(a_hbm, b_hbm, row_hbm, col_hbm, s_hbm,
                  rid_all, cid_all, abuf0, bbuf0, sbuf0, abuf1, bbuf1, sbuf1,
                  sema0, semb0, sems0, sema1, semb1, sems1):
        wid = lax.axis_index("s") * NC + lax.axis_index("c")
        base = wid * EPW
        pltpu.sync_copy(row_hbm.at[pl.ds(base, EPW)], rid_all)
        pltpu.sync_copy(col_hbm.at[pl.ds(base, EPW)], cid_all)

        def issue(m, abuf, bbuf, sema, semb):
            for i in range(SUB):
                sl = pl.ds(m * MEGA + i * K, K)
                dl = pl.ds(i * K, K)
                pltpu.async_copy(a_hbm.at[rid_all.at[sl]], abuf.at[dl], sema)
                pltpu.async_copy(b_hbm.at[cid_all.at[sl]], bbuf.at[dl], semb)

        def process(m, abuf, bbuf, sbuf, sema, semb, sems):
            pltpu.make_async_copy(a_hbm.at[pl.ds(0, MEGA)], abuf, sema).wait()
            pltpu.make_async_copy(b_hbm.at[pl.ds(0, MEGA)], bbuf, semb).wait()

            @pl.when(m >= 2)
            def _():
                pltpu.make_async_copy(s_hbm.at[pl.ds(0, MEGA)], sbuf,
                                      sems).wait()

            def add4(i, c2):
                for u in range(4):
                    r = i * 4 + u
                    for h in range(MD // 16):
                        sl = pl.ds(h * 16, 16)
                        sbuf[r, sl] = abuf[r, sl] + bbuf[r, sl]
                return c2

            lax.fori_loop(0, MEGA // 4, add4, 0)
            pltpu.async_copy(sbuf, s_hbm.at[pl.ds(base + m * MEGA, MEGA)],
                             sems)

        issue(0, abuf0, bbuf0, sema0, semb0)

        def body(jj, carry):
            @pl.when(jj % 2 == 0)
            def _():
                issue(jj + 1, abuf1, bbuf1, sema1, semb1)
                process(jj, abuf0, bbuf0, sbuf0, sema0, semb0, sems0)

            @pl.when(jj % 2 == 1)
            def _():
                issue(jj + 1, abuf0, bbuf0, sema0, semb0)
                process(jj, abuf1, bbuf1, sbuf1, sema1, semb1, sems1)

            return carry

        lax.fori_loop(0, NMEGA - 1, body, 0)
        process(NMEGA - 1, abuf0, bbuf0, sbuf0, sema0, semb0, sems0)
        pltpu.make_async_copy(s_hbm.at[pl.ds(0, MEGA)], sbuf1, sems1).wait()
        pltpu.make_async_copy(s_hbm.at[pl.ds(0, MEGA)], sbuf0, sems0).wait()

    return sc_gather


@functools.lru_cache(maxsize=None)
def _make_sc_scatter():
    mesh = plsc.VectorSubcoreMesh(core_axis_name="c", subcore_axis_name="s")

    @functools.partial(
        pl.kernel,
        mesh=mesh,
        compiler_params=pltpu.CompilerParams(use_tc_tiling_on_sc=False),
        out_type=jax.ShapeDtypeStruct((NC, N, MD), jnp.float32),
        scratch_types=[
            pltpu.VMEM((EPW,), jnp.int32),
            pltpu.VMEM((K,), jnp.int32),
            pltpu.VMEM((MEGA, MD), jnp.float32),
            pltpu.VMEM((MEGA, MD), jnp.float32),
            pltpu.VMEM((NPS, MD), jnp.float32),
            pltpu.VMEM_SHARED((N, MD), jnp.float32),
            pltpu.SemaphoreType.DMA,
            pltpu.SemaphoreType.DMA,
        ],
    )
    def sc_scatter(msg_hbm, row_hbm, aggp_hbm,
                   rid_all, ridk, mbuf0, mbuf1, zbuf, agg_sh, semm0, semm1):
        cidx = lax.axis_index("c")
        sidx = lax.axis_index("s")
        wid = sidx * NC + cidx
        base = wid * EPW

        pltpu.sync_copy(row_hbm.at[pl.ds(base, EPW)], rid_all)
        pltpu.async_copy(msg_hbm.at[pl.ds(base, MEGA)], mbuf0, semm0)

        zero = jnp.zeros((16,), jnp.float32)

        def zrow(i, c2):
            for u in range(4):
                for h in range(MD // 16):
                    zbuf[i * 4 + u, pl.ds(h * 16, 16)] = zero
            return c2

        lax.fori_loop(0, NPS // 4, zrow, 0)
        for h in range(MD // 16):
            zbuf[NPS - 1, pl.ds(h * 16, 16)] = zero
        pltpu.sync_copy(zbuf, agg_sh.at[pl.ds(sidx * NPS, NPS)])
        plsc.subcore_barrier()

        def process(m, mbuf, semm):
            pltpu.make_async_copy(msg_hbm.at[pl.ds(0, MEGA)], mbuf,
                                  semm).wait()
            for i in range(SUB):
                for h in range(K // 16):
                    ridk[pl.ds(h * 16, 16)] = rid_all[
                        pl.ds(m * MEGA + i * K + h * 16, 16)]
                pltpu.sync_copy(mbuf.at[pl.ds(i * K, K)],
                                agg_sh.at[ridk], add=True)

        def body(jj, carry):
            @pl.when(jj % 2 == 0)
            def _():
                pltpu.async_copy(msg_hbm.at[pl.ds(base + (jj + 1) * MEGA,
                                                  MEGA)], mbuf1, semm1)
                process(jj, mbuf0, semm0)

            @pl.when(jj % 2 == 1)
            def _():
                pltpu.async_copy(msg_hbm.at[pl.ds(base + (jj + 1) * MEGA,
                                                  MEGA)], mbuf0, semm0)
                process(jj, mbuf1, semm1)

            return carry

        lax.fori_loop(0, NMEGA - 1, body, 0)
        process(NMEGA - 1, mbuf0, semm0)
        plsc.subcore_barrier()
        pltpu.sync_copy(agg_sh.at[pl.ds(sidx * NPS, NPS)],
                        aggp_hbm.at[cidx, pl.ds(sidx * NPS, NPS)])

    return sc_scatter


def _temb_in_kernel(ts, wt1, bt1, wt2, bt2):
    h = ts * wt1 + bt1
    h = h * jax.nn.sigmoid(h)
    return jnp.dot(h, wt2, preferred_element_type=jnp.float32) + bt2


def _tc_pre_kernel(x_ref, ts_ref, wt1_ref, bt1_ref, wt2_ref, bt2_ref,
                   gx_ref, gt_ref, bx_ref, btc_ref, winx_ref, wint_ref,
                   w1a_ref, w1b_ref, b1_ref,
                   feats_ref, a_ref, b_ref):
    temb = _temb_in_kernel(ts_ref[...], wt1_ref[...], bt1_ref[...],
                           wt2_ref[...], bt2_ref[...])
    xb = x_ref[...]
    sx = jnp.sum(xb, axis=1, keepdims=True)
    st = jnp.sum(temb, axis=1, keepdims=True)
    mu = (sx + st) / float(D + MD)
    xc = xb - mu
    tc = temb - mu
    var = (jnp.sum(xc * xc, axis=1, keepdims=True)
           + jnp.sum(tc * tc, axis=1, keepdims=True)) / float(D + MD)
    rstd = lax.rsqrt(var + 1e-3)
    fx = xc * rstd * gx_ref[...] + bx_ref[...]
    ft = tc * rstd * gt_ref[...] + btc_ref[...]
    feats = (jnp.dot(fx, winx_ref[...], preferred_element_type=jnp.float32)
             + jnp.dot(ft, wint_ref[...], preferred_element_type=jnp.float32))
    feats_ref[...] = feats
    a_ref[...] = jnp.dot(feats, w1a_ref[...],
                         preferred_element_type=jnp.float32) + b1_ref[...]
    b_ref[...] = jnp.dot(feats, w1b_ref[...],
                         preferred_element_type=jnp.float32)


def _tc_pre(x, ts32, wt1, bt1, wt2, bt2, gx, gt, bx, btc, winx, wint,
            w1a, w1b, b1):
    full = lambda shp: pl.BlockSpec(shp, lambda i: (0,) * len(shp))
    return pl.pallas_call(
        _tc_pre_kernel,
        grid=(N // BN,),
        in_specs=[
            pl.BlockSpec((BN, D), lambda i: (i, 0)),
            full((1, MD)), full((1, MD)), full((1, MD)),
            full((MD, MD)), full((1, MD)),
            full((1, D)), full((1, MD)), full((1, D)), full((1, MD)),
            full((D, MD)), full((MD, MD)),
            full((MD, MD)), full((MD, MD)), full((1, MD)),
        ],
        out_specs=[
            pl.BlockSpec((BN, MD), lambda i: (i, 0)),
            pl.BlockSpec((BN, MD), lambda i: (i, 0)),
            pl.BlockSpec((BN, MD), lambda i: (i, 0)),
        ],
        out_shape=[
            jax.ShapeDtypeStruct((N, MD), jnp.float32),
            jax.ShapeDtypeStruct((N, MD), jnp.float32),
            jax.ShapeDtypeStruct((N, MD), jnp.float32),
        ],
    )(x, ts32, wt1, bt1, wt2, bt2, gx, gt, bx, btc, winx, wint, w1a, w1b, b1)


def _tc_msg_kernel(s_ref, w2_ref, b2_ref, msg_ref):
    m1 = jax.nn.sigmoid(s_ref[...])
    z = jnp.dot(m1, w2_ref[...], preferred_element_type=jnp.float32) + b2_ref[...]
    msg_ref[...] = _softsign(z)


def _tc_msg(s, w2, b2):
    full = lambda shp: pl.BlockSpec(shp, lambda i: (0,) * len(shp))
    return pl.pallas_call(
        _tc_msg_kernel,
        grid=(E // BE,),
        in_specs=[
            pl.BlockSpec((BE, MD), lambda i: (i, 0)),
            full((MD, MD)), full((1, MD)),
        ],
        out_specs=pl.BlockSpec((BE, MD), lambda i: (i, 0)),
        out_shape=jax.ShapeDtypeStruct((E, MD), jnp.float32),
    )(s, w2, b2)


def _tc_update_kernel(last, f_ref, aggp_ref, ts_ref, wt1_ref, bt1_ref,
                      wt2_ref, bt2_ref, fa_ref, fb_ref, fc_ref, fb1_ref,
                      fw2_ref, fb2_ref, wx_ref, wy_ref, by_ref, *out_refs):
    temb = _temb_in_kernel(ts_ref[...], wt1_ref[...], bt1_ref[...],
                           wt2_ref[...], bt2_ref[...])
    agg = aggp_ref[0] + aggp_ref[1]
    gs = jax.nn.sigmoid(f_ref[...])
    ga = jax.nn.sigmoid(agg)
    gt = jax.nn.sigmoid(temb)
    g1 = jax.nn.sigmoid(
        jnp.dot(gs, fa_ref[...], preferred_element_type=jnp.float32)
        + jnp.dot(ga, fb_ref[...], preferred_element_type=jnp.float32)
        + jnp.dot(gt, fc_ref[...], preferred_element_type=jnp.float32)
        + fb1_ref[...])
    f2 = _softsign(jnp.dot(g1, fw2_ref[...],
                           preferred_element_type=jnp.float32) + fb2_ref[...])
    if last:
        out_refs[0][...] = jnp.dot(f2, wx_ref[...],
                                   preferred_element_type=jnp.float32)
    else:
        out_refs[0][...] = f2
        out_refs[1][...] = jnp.dot(f2, wx_ref[...],
                                   preferred_element_type=jnp.float32) + by_ref[...]
        out_refs[2][...] = jnp.dot(f2, wy_ref[...],
                                   preferred_element_type=jnp.float32)


def _tc_update(last, feats, aggp, ts32, wt1, bt1, wt2, bt2,
               fa, fb, fc, fb1, fw2, fb2, wx, wy, by):
    full = lambda shp: pl.BlockSpec(shp, lambda i: (0,) * len(shp))
    if last:
        out_specs = [pl.BlockSpec((BN, DOUT), lambda i: (i, 0))]
        out_shape = [jax.ShapeDtypeStruct((N, DOUT), jnp.float32)]
        wx_spec = full((MD, DOUT))
    else:
        out_specs = [pl.BlockSpec((BN, MD), lambda i: (i, 0))] * 3
        out_shape = [jax.ShapeDtypeStruct((N, MD), jnp.float32)] * 3
        wx_spec = full((MD, MD))
    res = pl.pallas_call(
        functools.partial(_tc_update_kernel, last),
        grid=(N // BN,),
        in_specs=[
            pl.BlockSpec((BN, MD), lambda i: (i, 0)),
            pl.BlockSpec((NC, BN, MD), lambda i: (0, i, 0)),
            full((1, MD)), full((1, MD)), full((1, MD)),
            full((MD, MD)), full((1, MD)),
            full((MD, MD)), full((MD, MD)), full((MD, MD)), full((1, MD)),
            full((MD, MD)), full((1, MD)),
            wx_spec, full((MD, MD)), full((1, MD)),
        ],
        out_specs=out_specs,
        out_shape=out_shape,
    )(feats, aggp, ts32, wt1, bt1, wt2, bt2,
      fa, fb, fc, fb1, fw2, fb2, wx, wy, by)
    return res


def kernel(x, edge_index, time_step, Wt1, bt1, Wt2, bt2, gamma, beta, Win,
           msgW1, msgb1, msgW2, msgb2, featW1, featb1, featW2, featb2, Wout):
    ei = edge_index.astype(jnp.int32)
    row = ei[0]
    col = ei[1]
    ts32 = jnp.full((1, MD), time_step, jnp.float32)

    wt1 = Wt1.reshape(1, MD)
    bt1r = bt1.reshape(1, MD)
    bt2r = bt2.reshape(1, MD)
    gx = gamma[:D].reshape(1, D)
    gt = gamma[D:].reshape(1, MD)
    bx = beta[:D].reshape(1, D)
    btc = beta[D:].reshape(1, MD)
    winx = Win[:D]
    wint = Win[D:]

    w1a = [msgW1[i, :MD] for i in range(L)]
    w1b = [msgW1[i, MD:] for i in range(L)]
    b1 = [msgb1[i].reshape(1, MD) for i in range(L)]
    w2 = [msgW2[i] for i in range(L)]
    b2 = [msgb2[i].reshape(1, MD) for i in range(L)]
    fa = [featW1[i, :MD] for i in range(L)]
    fb = [featW1[i, MD:2 * MD] for i in range(L)]
    fc = [featW1[i, 2 * MD:] for i in range(L)]
    fb1 = [featb1[i].reshape(1, MD) for i in range(L)]
    fw2 = [featW2[i] for i in range(L)]
    fb2 = [featb2[i].reshape(1, MD) for i in range(L)]

    feats, A, B = _tc_pre(x, ts32, wt1, bt1r, Wt2, bt2r, gx, gt, bx, btc,
                          winx, wint, w1a[0], w1b[0], b1[0])

    sc_gather = _make_sc_gather()
    sc_scatter = _make_sc_scatter()
    out = None
    for i in range(L):
        s = sc_gather(A, B, row, col)
        msg = _tc_msg(s, w2[i], b2[i])
        aggp = sc_scatter(msg, row)
        last = i == L - 1
        if last:
            (out,) = _tc_update(True, feats, aggp, ts32, wt1, bt1r, Wt2,
                                bt2r, fa[i], fb[i], fc[i], fb1[i], fw2[i],
                                fb2[i], Wout, fw2[i], fb2[i])
        else:
            feats, A, B = _tc_update(False, feats, aggp, ts32, wt1, bt1r,
                                     Wt2, bt2r, fa[i], fb[i], fc[i], fb1[i],
                                     fw2[i], fb2[i], w1a[i + 1], w1b[i + 1],
                                     b1[i + 1])
    return out


# speedup vs baseline: 13.6905x; 3.0888x over previous
import functools

import jax
import jax.numpy as jnp
from jax import lax
from jax.experimental import pallas as pl
from jax.experimental.pallas import tpu as pltpu
from jax.experimental.pallas import tpu_sc as plsc

N = 10000
E = 320000
D = 128
MD = 32
L = 4
DOUT = 128

NC = 2
NS = 16
NW = NC * NS
EPW = E // NW
K = 80
MEGA = 400
SUB = MEGA // K
NMEGA = EPW // MEGA
NPS = N // NS

BN = 1000
E4 = E // 4
BE4 = 4000


def _softsign(v):
    return v / (1.0 + jnp.abs(v))


@functools.lru_cache(maxsize=None)
def _make_sc_gather():
    mesh = plsc.VectorSubcoreMesh(core_axis_name="c", subcore_axis_name="s")

    @functools.partial(
        pl.kernel,
        mesh=mesh,
        compiler_params=pltpu.CompilerParams(use_tc_tiling_on_sc=False),
        out_type=jax.ShapeDtypeStruct((E, MD), jnp.float32),
        scratch_types=[
            pltpu.VMEM((EPW,), jnp.int32),
            pltpu.VMEM((EPW,), jnp.int32),
            pltpu.VMEM((MEGA, MD), jnp.float32),
            pltpu.VMEM((MEGA, MD), jnp.float32),
            pltpu.VMEM((MEGA, MD), jnp.float32),
            pltpu.VMEM((MEGA, MD), jnp.float32),
            pltpu.VMEM((MEGA, MD), jnp.float32),
            pltpu.VMEM((MEGA, MD), jnp.float32),
            pltpu.SemaphoreType.DMA,
            pltpu.SemaphoreType.DMA,
            pltpu.SemaphoreType.DMA,
            pltpu.SemaphoreType.DMA,
            pltpu.SemaphoreType.DMA,
            pltpu.SemaphoreType.DMA,
        ],
    )
    def sc_gather(a_hbm, b_hbm, row_hbm, col_hbm, s_hbm,
                  rid_all, cid_all, abuf0, bbuf0, sbuf0, abuf1, bbuf1, sbuf1,
                  sema0, semb0, sems0, sema1, semb1, sems1):
        wid = lax.axis_index("s") * NC + lax.axis_index("c")
        base = wid * EPW
        pltpu.sync_copy(row_hbm.at[pl.ds(base, EPW)], rid_all)
        pltpu.sync_copy(col_hbm.at[pl.ds(base, EPW)], cid_all)

        def issue(m, abuf, bbuf, sema, semb):
            for i in range(SUB):
                sl = pl.ds(m * MEGA + i * K, K)
                dl = pl.ds(i * K, K)
                pltpu.async_copy(a_hbm.at[rid_all.at[sl]], abuf.at[dl], sema)
                pltpu.async_copy(b_hbm.at[cid_all.at[sl]], bbuf.at[dl], semb)

        def process(m, abuf, bbuf, sbuf, sema, semb, sems):
            pltpu.make_async_copy(a_hbm.at[pl.ds(0, MEGA)], abuf, sema).wait()
            pltpu.make_async_copy(b_hbm.at[pl.ds(0, MEGA)], bbuf, semb).wait()

            @pl.when(m >= 2)
            def _():
                pltpu.make_async_copy(s_hbm.at[pl.ds(0, MEGA)], sbuf,
                                      sems).wait()

            def add4(i, c2):
                for u in range(4):
                    r = i * 4 + u
                    for h in range(MD // 16):
                        sl = pl.ds(h * 16, 16)
                        sbuf[r, sl] = abuf[r, sl] + bbuf[r, sl]
                return c2

            lax.fori_loop(0, MEGA // 4, add4, 0)
            pltpu.async_copy(sbuf, s_hbm.at[pl.ds(base + m * MEGA, MEGA)],
                             sems)

        issue(0, abuf0, bbuf0, sema0, semb0)

        def body(jj, carry):
            @pl.when(jj % 2 == 0)
            def _():
                issue(jj + 1, abuf1, bbuf1, sema1, semb1)
                process(jj, abuf0, bbuf0, sbuf0, sema0, semb0, sems0)

            @pl.when(jj % 2 == 1)
            def _():
                issue(jj + 1, abuf0, bbuf0, sema0, semb0)
                process(jj, abuf1, bbuf1, sbuf1, sema1, semb1, sems1)

            return carry

        lax.fori_loop(0, NMEGA - 1, body, 0)
        process(NMEGA - 1, abuf0, bbuf0, sbuf0, sema0, semb0, sems0)
        pltpu.make_async_copy(s_hbm.at[pl.ds(0, MEGA)], sbuf1, sems1).wait()
        pltpu.make_async_copy(s_hbm.at[pl.ds(0, MEGA)], sbuf0, sems0).wait()

    return sc_gather


@functools.lru_cache(maxsize=None)
def _make_sc_scatter():
    mesh = plsc.VectorSubcoreMesh(core_axis_name="c", subcore_axis_name="s")

    @functools.partial(
        pl.kernel,
        mesh=mesh,
        compiler_params=pltpu.CompilerParams(use_tc_tiling_on_sc=False),
        out_type=jax.ShapeDtypeStruct((NC, N, MD), jnp.float32),
        scratch_types=[
            pltpu.VMEM((EPW,), jnp.int32),
            pltpu.VMEM((K,), jnp.int32),
            pltpu.VMEM((MEGA, MD), jnp.float32),
            pltpu.VMEM((MEGA, MD), jnp.float32),
            pltpu.VMEM((NPS, MD), jnp.float32),
            pltpu.VMEM_SHARED((N, MD), jnp.float32),
            pltpu.SemaphoreType.DMA,
            pltpu.SemaphoreType.DMA,
        ],
    )
    def sc_scatter(msg_hbm, row_hbm, aggp_hbm,
                   rid_all, ridk, mbuf0, mbuf1, zbuf, agg_sh, semm0, semm1):
        cidx = lax.axis_index("c")
        sidx = lax.axis_index("s")
        wid = sidx * NC + cidx
        base = wid * EPW

        pltpu.sync_copy(row_hbm.at[pl.ds(base, EPW)], rid_all)
        pltpu.async_copy(msg_hbm.at[pl.ds(base, MEGA)], mbuf0, semm0)

        zero = jnp.zeros((16,), jnp.float32)

        def zrow(i, c2):
            for u in range(4):
                for h in range(MD // 16):
                    zbuf[i * 4 + u, pl.ds(h * 16, 16)] = zero
            return c2

        lax.fori_loop(0, NPS // 4, zrow, 0)
        for h in range(MD // 16):
            zbuf[NPS - 1, pl.ds(h * 16, 16)] = zero
        pltpu.sync_copy(zbuf, agg_sh.at[pl.ds(sidx * NPS, NPS)])
        plsc.subcore_barrier()

        def process(m, mbuf, semm):
            pltpu.make_async_copy(msg_hbm.at[pl.ds(0, MEGA)], mbuf,
                                  semm).wait()
            for i in range(SUB):
                for h in range(K // 16):
                    ridk[pl.ds(h * 16, 16)] = rid_all[
                        pl.ds(m * MEGA + i * K + h * 16, 16)]
                pltpu.sync_copy(mbuf.at[pl.ds(i * K, K)],
                                agg_sh.at[ridk], add=True)

        def body(jj, carry):
            @pl.when(jj % 2 == 0)
            def _():
                pltpu.async_copy(msg_hbm.at[pl.ds(base + (jj + 1) * MEGA,
                                                  MEGA)], mbuf1, semm1)
                process(jj, mbuf0, semm0)

            @pl.when(jj % 2 == 1)
            def _():
                pltpu.async_copy(msg_hbm.at[pl.ds(base + (jj + 1) * MEGA,
                                                  MEGA)], mbuf0, semm0)
                process(jj, mbuf1, semm1)

            return carry

        lax.fori_loop(0, NMEGA - 1, body, 0)
        process(NMEGA - 1, mbuf0, semm0)
        plsc.subcore_barrier()
        pltpu.sync_copy(agg_sh.at[pl.ds(sidx * NPS, NPS)],
                        aggp_hbm.at[cidx, pl.ds(sidx * NPS, NPS)])

    return sc_scatter


def _temb_in_kernel(ts, wt1, bt1, wt2, bt2):
    h = ts * wt1 + bt1
    h = h * jax.nn.sigmoid(h)
    return jnp.dot(h, wt2, preferred_element_type=jnp.float32) + bt2


def _tc_pre_kernel(x_ref, ts_ref, wt1_ref, bt1_ref, wt2_ref, bt2_ref,
                   gx_ref, gt_ref, bx_ref, btc_ref, winx_ref, wint_ref,
                   w1a_ref, w1b_ref, b1_ref,
                   feats_ref, a_ref, b_ref):
    temb = _temb_in_kernel(ts_ref[...], wt1_ref[...], bt1_ref[...],
                           wt2_ref[...], bt2_ref[...])
    xb = x_ref[...]
    sx = jnp.sum(xb, axis=1, keepdims=True)
    st = jnp.sum(temb, axis=1, keepdims=True)
    mu = (sx + st) / float(D + MD)
    xc = xb - mu
    tc = temb - mu
    var = (jnp.sum(xc * xc, axis=1, keepdims=True)
           + jnp.sum(tc * tc, axis=1, keepdims=True)) / float(D + MD)
    rstd = lax.rsqrt(var + 1e-3)
    fx = xc * rstd * gx_ref[...] + bx_ref[...]
    ft = tc * rstd * gt_ref[...] + btc_ref[...]
    feats = (jnp.dot(fx, winx_ref[...], preferred_element_type=jnp.float32)
             + jnp.dot(ft, wint_ref[...], preferred_element_type=jnp.float32))
    feats_ref[...] = feats
    a_ref[...] = jnp.dot(feats, w1a_ref[...],
                         preferred_element_type=jnp.float32) + b1_ref[...]
    b_ref[...] = jnp.dot(feats, w1b_ref[...],
                         preferred_element_type=jnp.float32)


def _tc_pre(x, ts32, wt1, bt1, wt2, bt2, gx, gt, bx, btc, winx, wint,
            w1a, w1b, b1):
    full = lambda shp: pl.BlockSpec(shp, lambda i: (0,) * len(shp))
    return pl.pallas_call(
        _tc_pre_kernel,
        grid=(N // BN,),
        in_specs=[
            pl.BlockSpec((BN, D), lambda i: (i, 0)),
            full((1, MD)), full((1, MD)), full((1, MD)),
            full((MD, MD)), full((1, MD)),
            full((1, D)), full((1, MD)), full((1, D)), full((1, MD)),
            full((D, MD)), full((MD, MD)),
            full((MD, MD)), full((MD, MD)), full((1, MD)),
        ],
        out_specs=[
            pl.BlockSpec((BN, MD), lambda i: (i, 0)),
            pl.BlockSpec((BN, MD), lambda i: (i, 0)),
            pl.BlockSpec((BN, MD), lambda i: (i, 0)),
        ],
        out_shape=[
            jax.ShapeDtypeStruct((N, MD), jnp.float32),
            jax.ShapeDtypeStruct((N, MD), jnp.float32),
            jax.ShapeDtypeStruct((N, MD), jnp.float32),
        ],
    )(x, ts32, wt1, bt1, wt2, bt2, gx, gt, bx, btc, winx, wint, w1a, w1b, b1)


def _tc_msg_kernel(s_ref, w2_ref, b2_ref, msg_ref):
    m1 = jax.nn.sigmoid(s_ref[...])
    z = jnp.dot(m1, w2_ref[...], preferred_element_type=jnp.float32) + b2_ref[...]
    msg_ref[...] = _softsign(z)


def _tc_msg(s, w2bd, b2t):
    full = lambda shp: pl.BlockSpec(shp, lambda i: (0,) * len(shp))
    s4 = s.reshape(E4, 4 * MD)
    msg4 = pl.pallas_call(
        _tc_msg_kernel,
        grid=(E4 // BE4,),
        in_specs=[
            pl.BlockSpec((BE4, 4 * MD), lambda i: (i, 0)),
            full((4 * MD, 4 * MD)), full((1, 4 * MD)),
        ],
        out_specs=pl.BlockSpec((BE4, 4 * MD), lambda i: (i, 0)),
        out_shape=jax.ShapeDtypeStruct((E4, 4 * MD), jnp.float32),
    )(s4, w2bd, b2t)
    return msg4.reshape(E, MD)


def _tc_update_kernel(last, f_ref, aggp_ref, ts_ref, wt1_ref, bt1_ref,
                      wt2_ref, bt2_ref, fa_ref, fb_ref, fc_ref, fb1_ref,
                      fw2_ref, fb2_ref, wx_ref, wy_ref, by_ref, *out_refs):
    temb = _temb_in_kernel(ts_ref[...], wt1_ref[...], bt1_ref[...],
                           wt2_ref[...], bt2_ref[...])
    agg = aggp_ref[0] + aggp_ref[1]
    gs = jax.nn.sigmoid(f_ref[...])
    ga = jax.nn.sigmoid(agg)
    gt = jax.nn.sigmoid(temb)
    g1 = jax.nn.sigmoid(
        jnp.dot(gs, fa_ref[...], preferred_element_type=jnp.float32)
        + jnp.dot(ga, fb_ref[...], preferred_element_type=jnp.float32)
        + jnp.dot(gt, fc_ref[...], preferred_element_type=jnp.float32)
        + fb1_ref[...])
    f2 = _softsign(jnp.dot(g1, fw2_ref[...],
                           preferred_element_type=jnp.float32) + fb2_ref[...])
    if last:
        out_refs[0][...] = jnp.dot(f2, wx_ref[...],
                                   preferred_element_type=jnp.float32)
    else:
        out_refs[0][...] = f2
        out_refs[1][...] = jnp.dot(f2, wx_ref[...],
                                   preferred_element_type=jnp.float32) + by_ref[...]
        out_refs[2][...] = jnp.dot(f2, wy_ref[...],
                                   preferred_element_type=jnp.float32)


def _tc_update(last, feats, aggp, ts32, wt1, bt1, wt2, bt2,
               fa, fb, fc, fb1, fw2, fb2, wx, wy, by):
    full = lambda shp: pl.BlockSpec(shp, lambda i: (0,) * len(shp))
    if last:
        out_specs = [pl.BlockSpec((BN, DOUT), lambda i: (i, 0))]
        out_shape = [jax.ShapeDtypeStruct((N, DOUT), jnp.float32)]
        wx_spec = full((MD, DOUT))
    else:
        out_specs = [pl.BlockSpec((BN, MD), lambda i: (i, 0))] * 3
        out_shape = [jax.ShapeDtypeStruct((N, MD), jnp.float32)] * 3
        wx_spec = full((MD, MD))
    res = pl.pallas_call(
        functools.partial(_tc_update_kernel, last),
        grid=(N // BN,),
        in_specs=[
            pl.BlockSpec((BN, MD), lambda i: (i, 0)),
            pl.BlockSpec((NC, BN, MD), lambda i: (0, i, 0)),
            full((1, MD)), full((1, MD)), full((1, MD)),
            full((MD, MD)), full((1, MD)),
            full((MD, MD)), full((MD, MD)), full((MD, MD)), full((1, MD)),
            full((MD, MD)), full((1, MD)),
            wx_spec, full((MD, MD)), full((1, MD)),
        ],
        out_specs=out_specs,
        out_shape=out_shape,
    )(feats, aggp, ts32, wt1, bt1, wt2, bt2,
      fa, fb, fc, fb1, fw2, fb2, wx, wy, by)
    return res


def kernel(x, edge_index, time_step, Wt1, bt1, Wt2, bt2, gamma, beta, Win,
           msgW1, msgb1, msgW2, msgb2, featW1, featb1, featW2, featb2, Wout):
    ei = edge_index.astype(jnp.int32)
    row = ei[0]
    col = ei[1]
    ts32 = jnp.full((1, MD), time_step, jnp.float32)

    wt1 = Wt1.reshape(1, MD)
    bt1r = bt1.reshape(1, MD)
    bt2r = bt2.reshape(1, MD)
    gx = gamma[:D].reshape(1, D)
    gt = gamma[D:].reshape(1, MD)
    bx = beta[:D].reshape(1, D)
    btc = beta[D:].reshape(1, MD)
    winx = Win[:D]
    wint = Win[D:]

    w1a = [msgW1[i, :MD] for i in range(L)]
    w1b = [msgW1[i, MD:] for i in range(L)]
    b1 = [msgb1[i].reshape(1, MD) for i in range(L)]
    eye4 = jnp.eye(4, dtype=jnp.float32)
    w2 = [jnp.kron(eye4, msgW2[i]) for i in range(L)]
    b2 = [jnp.tile(msgb2[i].reshape(1, MD), (1, 4)) for i in range(L)]
    fa = [featW1[i, :MD] for i in range(L)]
    fb = [featW1[i, MD:2 * MD] for i in range(L)]
    fc = [featW1[i, 2 * MD:] for i in range(L)]
    fb1 = [featb1[i].reshape(1, MD) for i in range(L)]
    fw2 = [featW2[i] for i in range(L)]
    fb2 = [featb2[i].reshape(1, MD) for i in range(L)]

    feats, A, B = _tc_pre(x, ts32, wt1, bt1r, Wt2, bt2r, gx, gt, bx, btc,
                          winx, wint, w1a[0], w1b[0], b1[0])

    sc_gather = _make_sc_gather()
    sc_scatter = _make_sc_scatter()
    out = None
    for i in range(L):
        s = sc_gather(A, B, row, col)
        msg = _tc_msg(s, w2[i], b2[i])
        aggp = sc_scatter(msg, row)
        last = i == L - 1
        if last:
            (out,) = _tc_update(True, feats, aggp, ts32, wt1, bt1r, Wt2,
                                bt2r, fa[i], fb[i], fc[i], fb1[i], fw2[i],
                                fb2[i], Wout, fw2[i], fb2[i])
        else:
            feats, A, B = _tc_update(False, feats, aggp, ts32, wt1, bt1r,
                                     Wt2, bt2r, fa[i], fb[i], fc[i], fb1[i],
                                     fw2[i], fb2[i], w1a[i + 1], w1b[i + 1],
                                     b1[i + 1])
    return out
